# baseline (speedup 1.0000x reference)
import jax
import jax.numpy as jnp
from jax.experimental import pallas as pl


def _add_kernel(x_ref, e_ref, o_ref):
    o_ref[...] = x_ref[...] + e_ref[...]


def kernel(inputs, embeddings):
    B, S, D = inputs.shape
    BS = 512
    x2 = inputs.reshape(B * S, D)
    n_eb = S // BS
    out = pl.pallas_call(
        _add_kernel,
        grid=(B * S // BS,),
        in_specs=[
            pl.BlockSpec((BS, D), lambda i: (i, 0)),
            pl.BlockSpec((BS, D), lambda i: (i % n_eb, 0)),
        ],
        out_specs=pl.BlockSpec((BS, D), lambda i: (i, 0)),
        out_shape=jax.ShapeDtypeStruct((B * S, D), inputs.dtype),
    )(x2, embeddings[:S])
    return out.reshape(B, S, D)


# speedup vs baseline: 1.3622x; 1.3622x over previous
import jax
import jax.numpy as jnp
from jax.experimental import pallas as pl


def _add_kernel(x_ref, e_ref, o_ref):
    o_ref[...] = x_ref[...] + e_ref[...][None, :, :]


def kernel(inputs, embeddings):
    B, S, D = inputs.shape
    BS = 2048
    return pl.pallas_call(
        _add_kernel,
        grid=(S // BS, B),
        in_specs=[
            pl.BlockSpec((1, BS, D), lambda i, b: (b, i, 0)),
            pl.BlockSpec((BS, D), lambda i, b: (i, 0)),
        ],
        out_specs=pl.BlockSpec((1, BS, D), lambda i, b: (b, i, 0)),
        out_shape=jax.ShapeDtypeStruct((B, S, D), inputs.dtype),
    )(inputs, embeddings[:S])
